# baseline (speedup 1.0000x reference)
import jax
import jax.numpy as jnp
from jax.experimental import pallas as pl
from jax.experimental.pallas import tpu as pltpu

_MAP = (4, 4, 8, 7, 7, 10, 9, 9, 0, 1, 1, 2, 3, 3, 5, 6, 5, 6)

_N, _D, _T, _P = 64, 256, 300, 11
_J = 18
_BD = 32


def _gather_block(x_ref, o_ref):
    x = x_ref[...]
    o_ref[...] = jnp.concatenate([x[..., m:m + 1] for m in _MAP], axis=-1)


def kernel(part):
    grid = (_N, _D // _BD)
    return pl.pallas_call(
        _gather_block,
        grid=grid,
        in_specs=[pl.BlockSpec((1, _BD, _T, _P), lambda n, d: (n, d, 0, 0))],
        out_specs=pl.BlockSpec((1, _BD, _T, _J), lambda n, d: (n, d, 0, 0)),
        out_shape=jax.ShapeDtypeStruct((_N, _D, _T, _J), part.dtype),
        compiler_params=pltpu.CompilerParams(
            dimension_semantics=("arbitrary", "arbitrary"),
        ),
    )(part)


# speedup vs baseline: 3.5003x; 3.5003x over previous
import jax
import jax.numpy as jnp
import numpy as np
from jax.experimental import pallas as pl
from jax.experimental.pallas import tpu as pltpu

_MAP = (4, 4, 8, 7, 7, 10, 9, 9, 0, 1, 1, 2, 3, 3, 5, 6, 5, 6)

_N, _D, _T, _P = 64, 256, 300, 11
_J = 18
_BR = 32

_SEL = np.zeros((_P, _J), dtype=np.float32)
for _j, _m in enumerate(_MAP):
    _SEL[_m, _j] = 1.0


def _gather_block(sel_ref, x_ref, o_ref):
    sel = sel_ref[...]

    def body(i, _):
        x = x_ref[i].astype(jnp.bfloat16)
        o_ref[i] = jnp.dot(x, sel, preferred_element_type=jnp.float32)
        return 0

    jax.lax.fori_loop(0, _BR, body, 0, unroll=True)


def kernel(part):
    flat = part.reshape(_N * _D, _T, _P)
    sel = jnp.asarray(_SEL, dtype=jnp.bfloat16)
    out = pl.pallas_call(
        _gather_block,
        grid=(_N * _D // _BR,),
        in_specs=[
            pl.BlockSpec((_P, _J), lambda g: (0, 0)),
            pl.BlockSpec((_BR, _T, _P), lambda g: (g, 0, 0)),
        ],
        out_specs=pl.BlockSpec((_BR, _T, _J), lambda g: (g, 0, 0)),
        out_shape=jax.ShapeDtypeStruct((_N * _D, _T, _J), part.dtype),
        compiler_params=pltpu.CompilerParams(
            dimension_semantics=("arbitrary",),
        ),
    )(sel, flat)
    return out.reshape(_N, _D, _T, _J)
